# baseline (speedup 1.0000x reference)
import functools

import jax
import jax.numpy as jnp
from jax import lax
from jax.experimental import pallas as pl
from jax.experimental.pallas import tpu as pltpu
from jax.experimental.pallas import tpu_sc as plsc

NC = 2
NS = 16
NW = NC * NS
IPR = 128
IDXCH = 1024
W = 128


@functools.lru_cache(maxsize=None)
def _make_agg(n_nodes, e_pad):
    total_chunks = e_pad // IDXCH
    cpt = total_chunks // NW
    idx_rows = IDXCH // IPR
    NB = 4
    UR = 64
    upc = IDXCH // UR
    slab = (-(-(n_nodes + 1) // NS) + 7) // 8 * 8
    np_rows = NS * slab
    zr = slab
    mesh = plsc.VectorSubcoreMesh(core_axis_name="c", subcore_axis_name="s")

    @functools.partial(
        pl.kernel,
        out_type=jax.ShapeDtypeStruct((NC * np_rows, W), jnp.float32),
        mesh=mesh,
        scratch_types=[
            pltpu.VMEM((2, idx_rows, IPR), jnp.int32),
            pltpu.VMEM((2, idx_rows, IPR), jnp.int32),
            pltpu.VMEM((NB, UR, W), jnp.float32),
            pltpu.VMEM_SHARED((np_rows, W), jnp.float32),
            pltpu.SemaphoreType.DMA,
            pltpu.SemaphoreType.DMA,
            pltpu.SemaphoreType.DMA,
            pltpu.SemaphoreType.DMA,
            pltpu.SemaphoreType.DMA,
            pltpu.SemaphoreType.DMA,
            pltpu.SemaphoreType.DMA,
            pltpu.SemaphoreType.DMA,
            pltpu.SemaphoreType.DMA,
            pltpu.SemaphoreType.DMA,
        ],
    )
    def agg(src_hbm, dst_hbm, y_hbm, out_hbm, srcv, dstv, bufs, acc,
            is0, is1, gs0, gs1, gs2, gs3, ss0, ss1, ss2, ss3):
        isems = (is0, is1)
        gsems = (gs0, gs1, gs2, gs3)
        ssems = (ss0, ss1, ss2, ss3)
        c = lax.axis_index("c")
        s = lax.axis_index("s")

        zvec = jnp.zeros((16,), jnp.float32)

        def zrow(i, carry):
            for k in range(W // 16):
                bufs[0, i, pl.ds(k * 16, 16)] = zvec
            return carry

        lax.fori_loop(0, UR, zrow, 0)
        r0 = s * zr
        off = 0
        while off < zr:
            step = min(UR, zr - off)
            pltpu.sync_copy(bufs.at[0, pl.ds(0, step)],
                            acc.at[pl.ds(r0 + off, step)])
            off += step
        plsc.subcore_barrier()

        row_base = (c * NS + s) * cpt * idx_rows

        def idx_fetch(i, k):
            rb = row_base + i * idx_rows
            return (pltpu.async_copy(src_hbm.at[pl.ds(rb, idx_rows)],
                                     srcv.at[k], isems[k]),
                    pltpu.async_copy(dst_hbm.at[pl.ds(rb, idx_rows)],
                                     dstv.at[k], isems[k]))

        def iview(ref, k, u):
            return ref.at[k, u // 2, pl.ds((u % 2) * UR, UR)]

        pend = idx_fetch(0, 0)
        for i in range(cpt):
            k = i % 2
            pend[0].wait()
            pend[1].wait()
            if i + 1 < cpt:
                pend = idx_fetch(i + 1, (i + 1) % 2)
            gcp = {}
            scp = {}
            for u in range(NB):
                gcp[u] = pltpu.async_copy(y_hbm.at[iview(srcv, k, u)],
                                          bufs.at[u], gsems[u])
            for u in range(upc):
                b = u % NB
                gcp[u].wait()
                scp[u] = pltpu.async_copy(bufs.at[b],
                                          acc.at[iview(dstv, k, u)],
                                          ssems[b], add=True)
                nxt = u + 1
                if NB <= nxt < upc:
                    scp[nxt - NB].wait()
                    gcp[nxt] = pltpu.async_copy(y_hbm.at[iview(srcv, k, nxt)],
                                                bufs.at[nxt % NB],
                                                gsems[nxt % NB])
            for u in range(upc - NB, upc):
                scp[u].wait()
        plsc.subcore_barrier()

        pltpu.sync_copy(acc.at[pl.ds(r0, zr)],
                        out_hbm.at[pl.ds(c * np_rows + r0, zr)])

    return agg, np_rows


def _prep_body(x_ref, wcat_ref, y_ref, r_ref):
    n = y_ref.shape[0]
    hid = r_ref.shape[1]
    out = lax.dot_general(x_ref[...], wcat_ref[...],
                          (((1,), (0,)), ((), ())),
                          preferred_element_type=jnp.float32)
    col = lax.broadcasted_iota(jnp.int32, (n, W), 1)
    y_ref[...] = out[:, :W] + jnp.where(col == hid, 1.0, 0.0)
    r_ref[...] = out[:, W:]


def _mid_body(pa_ref, pb_ref, r_ref, b1_ref, g1_ref, be1_ref, w2_ref,
              yr2_ref, ci_ref):
    hid = r_ref.shape[1]
    p = pa_ref[...] + pb_ref[...]
    ssum = p[:, :hid]
    cnt = p[:, hid:hid + 1]
    cclip = jnp.maximum(cnt, 1.0)
    h = ssum / cclip + b1_ref[...][None, :] + r_ref[...]
    mu = jnp.mean(h, axis=0, keepdims=True)
    var = jnp.mean((h - mu) ** 2, axis=0, keepdims=True)
    hn = jnp.maximum(
        g1_ref[...][None, :] * (h - mu) / jnp.sqrt(var + 1e-5)
        + be1_ref[...][None, :], 0.0)
    yr2_ref[...] = lax.dot_general(hn, w2_ref[...], (((1,), (0,)), ((), ())),
                                   preferred_element_type=jnp.float32)
    ci_ref[...] = cclip


def _fin_body(pa_ref, pb_ref, yr2_ref, ci_ref, b2_ref, g2_ref, be2_ref,
              o_ref):
    hid = o_ref.shape[1]
    ssum = pa_ref[:, :hid] + pb_ref[:, :hid]
    r2 = yr2_ref[...][:, hid:]
    h = ssum / ci_ref[...] + b2_ref[...][None, :] + r2
    mu = jnp.mean(h, axis=0, keepdims=True)
    var = jnp.mean((h - mu) ** 2, axis=0, keepdims=True)
    hn = jnp.maximum(
        g2_ref[...][None, :] * (h - mu) / jnp.sqrt(var + 1e-5)
        + be2_ref[...][None, :], 0.0)
    o_ref[...] = jnp.max(hn, axis=0, keepdims=True)


def kernel(x, edge_index, Wl1, Wr1, b1, Wl2, Wr2, b2, g1, beta1, g2, beta2):
    n, in_dim = x.shape
    hid = Wl1.shape[0]
    e = edge_index.shape[1]

    e_pad = -(-e // (NW * IDXCH)) * (NW * IDXCH)
    pad = e_pad - e
    src = jnp.concatenate(
        [edge_index[0], jnp.zeros((pad,), jnp.int32)]).reshape(-1, IPR)
    dst = jnp.concatenate(
        [edge_index[1], jnp.full((pad,), n, jnp.int32)]).reshape(-1, IPR)

    w1cat = jnp.concatenate(
        [Wl1.T, jnp.zeros((in_dim, W - hid), jnp.float32), Wr1.T], axis=1)
    yaug, r1 = pl.pallas_call(
        _prep_body,
        out_shape=[jax.ShapeDtypeStruct((n, W), jnp.float32),
                   jax.ShapeDtypeStruct((n, hid), jnp.float32)],
    )(x, w1cat)

    agg, np_rows = _make_agg(n, e_pad)
    part1 = agg(src, dst, yaug)
    p1a = lax.slice(part1, (0, 0), (n, W))
    p1b = lax.slice(part1, (np_rows, 0), (np_rows + n, W))

    w2cat = jnp.concatenate([Wl2.T, Wr2.T], axis=1)
    yr2, ci = pl.pallas_call(
        _mid_body,
        out_shape=[jax.ShapeDtypeStruct((n, W), jnp.float32),
                   jax.ShapeDtypeStruct((n, 1), jnp.float32)],
    )(p1a, p1b, r1, b1, g1, beta1, w2cat)

    part2 = agg(src, dst, yr2)
    p2a = lax.slice(part2, (0, 0), (n, W))
    p2b = lax.slice(part2, (np_rows, 0), (np_rows + n, W))

    o = pl.pallas_call(
        _fin_body,
        out_shape=jax.ShapeDtypeStruct((1, hid), jnp.float32),
    )(p2a, p2b, yr2, ci, b2, g2, beta2)
    return o.reshape((hid,))


# speedup vs baseline: 1.0217x; 1.0217x over previous
import functools

import jax
import jax.numpy as jnp
from jax import lax
from jax.experimental import pallas as pl
from jax.experimental.pallas import tpu as pltpu
from jax.experimental.pallas import tpu_sc as plsc

NC = 2
NS = 16
NW = NC * NS
IPR = 128
IDXCH = 1024
W = 128


@functools.lru_cache(maxsize=None)
def _make_agg(n_nodes, e_pad):
    total_chunks = e_pad // IDXCH
    cpt = total_chunks // NW
    idx_rows = IDXCH // IPR
    NB = 2
    UR = 128
    upc = IDXCH // UR
    slab = (-(-(n_nodes + 1) // NS) + 7) // 8 * 8
    np_rows = NS * slab
    zr = slab
    mesh = plsc.VectorSubcoreMesh(core_axis_name="c", subcore_axis_name="s")

    @functools.partial(
        pl.kernel,
        out_type=jax.ShapeDtypeStruct((NC * np_rows, W), jnp.float32),
        mesh=mesh,
        scratch_types=[
            pltpu.VMEM((2, idx_rows, IPR), jnp.int32),
            pltpu.VMEM((2, idx_rows, IPR), jnp.int32),
            pltpu.VMEM((NB, UR, W), jnp.float32),
            pltpu.VMEM_SHARED((np_rows, W), jnp.float32),
            pltpu.SemaphoreType.DMA,
            pltpu.SemaphoreType.DMA,
            pltpu.SemaphoreType.DMA,
            pltpu.SemaphoreType.DMA,
            pltpu.SemaphoreType.DMA,
            pltpu.SemaphoreType.DMA,
            pltpu.SemaphoreType.DMA,
            pltpu.SemaphoreType.DMA,
            pltpu.SemaphoreType.DMA,
            pltpu.SemaphoreType.DMA,
        ],
    )
    def agg(src_hbm, dst_hbm, y_hbm, out_hbm, srcv, dstv, bufs, acc,
            is0, is1, gs0, gs1, gs2, gs3, ss0, ss1, ss2, ss3):
        isems = (is0, is1)
        gsems = (gs0, gs1, gs2, gs3)
        ssems = (ss0, ss1, ss2, ss3)
        c = lax.axis_index("c")
        s = lax.axis_index("s")

        zvec = jnp.zeros((16,), jnp.float32)

        def zrow(i, carry):
            for k in range(W // 16):
                bufs[0, i, pl.ds(k * 16, 16)] = zvec
            return carry

        lax.fori_loop(0, UR, zrow, 0)
        r0 = s * zr
        off = 0
        while off < zr:
            step = min(UR, zr - off)
            pltpu.sync_copy(bufs.at[0, pl.ds(0, step)],
                            acc.at[pl.ds(r0 + off, step)])
            off += step
        plsc.subcore_barrier()

        row_base = (c * NS + s) * cpt * idx_rows

        def idx_fetch(i, k):
            rb = row_base + i * idx_rows
            return (pltpu.async_copy(src_hbm.at[pl.ds(rb, idx_rows)],
                                     srcv.at[k], isems[k]),
                    pltpu.async_copy(dst_hbm.at[pl.ds(rb, idx_rows)],
                                     dstv.at[k], isems[k]))

        def iview(ref, k, u):
            if UR == IPR:
                return ref.at[k, u]
            return ref.at[k, u // 2, pl.ds((u % 2) * UR, UR)]

        pend = idx_fetch(0, 0)
        for i in range(cpt):
            k = i % 2
            pend[0].wait()
            pend[1].wait()
            if i + 1 < cpt:
                pend = idx_fetch(i + 1, (i + 1) % 2)
            gcp = {}
            scp = {}
            for u in range(NB):
                gcp[u] = pltpu.async_copy(y_hbm.at[iview(srcv, k, u)],
                                          bufs.at[u], gsems[u])
            for u in range(upc):
                b = u % NB
                gcp[u].wait()
                scp[u] = pltpu.async_copy(bufs.at[b],
                                          acc.at[iview(dstv, k, u)],
                                          ssems[b], add=True)
                nxt = u + 1
                if NB <= nxt < upc:
                    scp[nxt - NB].wait()
                    gcp[nxt] = pltpu.async_copy(y_hbm.at[iview(srcv, k, nxt)],
                                                bufs.at[nxt % NB],
                                                gsems[nxt % NB])
            for u in range(upc - NB, upc):
                scp[u].wait()
        plsc.subcore_barrier()

        pltpu.sync_copy(acc.at[pl.ds(r0, zr)],
                        out_hbm.at[pl.ds(c * np_rows + r0, zr)])

    return agg, np_rows


def _prep_body(x_ref, wcat_ref, y_ref, r_ref):
    n = y_ref.shape[0]
    hid = r_ref.shape[1]
    out = lax.dot_general(x_ref[...], wcat_ref[...],
                          (((1,), (0,)), ((), ())),
                          preferred_element_type=jnp.float32)
    col = lax.broadcasted_iota(jnp.int32, (n, W), 1)
    y_ref[...] = out[:, :W] + jnp.where(col == hid, 1.0, 0.0)
    r_ref[...] = out[:, W:]


def _mid_body(pa_ref, pb_ref, r_ref, b1_ref, g1_ref, be1_ref, w2_ref,
              yr2_ref, ci_ref):
    hid = r_ref.shape[1]
    p = pa_ref[...] + pb_ref[...]
    ssum = p[:, :hid]
    cnt = p[:, hid:hid + 1]
    cclip = jnp.maximum(cnt, 1.0)
    h = ssum / cclip + b1_ref[...][None, :] + r_ref[...]
    mu = jnp.mean(h, axis=0, keepdims=True)
    var = jnp.mean((h - mu) ** 2, axis=0, keepdims=True)
    hn = jnp.maximum(
        g1_ref[...][None, :] * (h - mu) / jnp.sqrt(var + 1e-5)
        + be1_ref[...][None, :], 0.0)
    yr2_ref[...] = lax.dot_general(hn, w2_ref[...], (((1,), (0,)), ((), ())),
                                   preferred_element_type=jnp.float32)
    ci_ref[...] = cclip


def _fin_body(pa_ref, pb_ref, yr2_ref, ci_ref, b2_ref, g2_ref, be2_ref,
              o_ref):
    hid = o_ref.shape[1]
    ssum = pa_ref[:, :hid] + pb_ref[:, :hid]
    r2 = yr2_ref[...][:, hid:]
    h = ssum / ci_ref[...] + b2_ref[...][None, :] + r2
    mu = jnp.mean(h, axis=0, keepdims=True)
    var = jnp.mean((h - mu) ** 2, axis=0, keepdims=True)
    hn = jnp.maximum(
        g2_ref[...][None, :] * (h - mu) / jnp.sqrt(var + 1e-5)
        + be2_ref[...][None, :], 0.0)
    o_ref[...] = jnp.max(hn, axis=0, keepdims=True)


def kernel(x, edge_index, Wl1, Wr1, b1, Wl2, Wr2, b2, g1, beta1, g2, beta2):
    n, in_dim = x.shape
    hid = Wl1.shape[0]
    e = edge_index.shape[1]

    e_pad = -(-e // (NW * IDXCH)) * (NW * IDXCH)
    pad = e_pad - e
    src = jnp.concatenate(
        [edge_index[0], jnp.zeros((pad,), jnp.int32)]).reshape(-1, IPR)
    dst = jnp.concatenate(
        [edge_index[1], jnp.full((pad,), n, jnp.int32)]).reshape(-1, IPR)

    w1cat = jnp.concatenate(
        [Wl1.T, jnp.zeros((in_dim, W - hid), jnp.float32), Wr1.T], axis=1)
    yaug, r1 = pl.pallas_call(
        _prep_body,
        out_shape=[jax.ShapeDtypeStruct((n, W), jnp.float32),
                   jax.ShapeDtypeStruct((n, hid), jnp.float32)],
    )(x, w1cat)

    agg, np_rows = _make_agg(n, e_pad)
    part1 = agg(src, dst, yaug)
    p1a = lax.slice(part1, (0, 0), (n, W))
    p1b = lax.slice(part1, (np_rows, 0), (np_rows + n, W))

    w2cat = jnp.concatenate([Wl2.T, Wr2.T], axis=1)
    yr2, ci = pl.pallas_call(
        _mid_body,
        out_shape=[jax.ShapeDtypeStruct((n, W), jnp.float32),
                   jax.ShapeDtypeStruct((n, 1), jnp.float32)],
    )(p1a, p1b, r1, b1, g1, beta1, w2cat)

    part2 = agg(src, dst, yr2)
    p2a = lax.slice(part2, (0, 0), (n, W))
    p2b = lax.slice(part2, (np_rows, 0), (np_rows + n, W))

    o = pl.pallas_call(
        _fin_body,
        out_shape=jax.ShapeDtypeStruct((1, hid), jnp.float32),
    )(p2a, p2b, yr2, ci, b2, g2, beta2)
    return o.reshape((hid,))


# speedup vs baseline: 1.0775x; 1.0546x over previous
import functools

import jax
import jax.numpy as jnp
from jax import lax
from jax.experimental import pallas as pl
from jax.experimental.pallas import tpu as pltpu
from jax.experimental.pallas import tpu_sc as plsc

NC = 2
NS = 16
NW = NC * NS
IPR = 128
IDXCH = 2048
W = 128


@functools.lru_cache(maxsize=None)
def _make_agg(n_nodes, e_pad):
    total_chunks = e_pad // IDXCH
    cpt = total_chunks // NW
    idx_rows = IDXCH // IPR
    NB = 2
    UR = 128
    upc = IDXCH // UR
    slab = (-(-(n_nodes + 1) // NS) + 7) // 8 * 8
    np_rows = NS * slab
    zr = slab
    mesh = plsc.VectorSubcoreMesh(core_axis_name="c", subcore_axis_name="s")

    @functools.partial(
        pl.kernel,
        out_type=jax.ShapeDtypeStruct((NC * np_rows, W), jnp.float32),
        mesh=mesh,
        scratch_types=[
            pltpu.VMEM((idx_rows, IPR), jnp.int32),
            pltpu.VMEM((idx_rows, IPR), jnp.int32),
            pltpu.VMEM((NB, UR, W), jnp.float32),
            pltpu.VMEM_SHARED((np_rows, W), jnp.float32),
            pltpu.SemaphoreType.DMA,
            pltpu.SemaphoreType.DMA,
            pltpu.SemaphoreType.DMA,
            pltpu.SemaphoreType.DMA,
        ],
    )
    def agg(src_hbm, dst_hbm, y_hbm, out_hbm, srcv, dstv, bufs, acc,
            gs0, gs1, ss0, ss1):
        gsems = (gs0, gs1)
        ssems = (ss0, ss1)
        c = lax.axis_index("c")
        s = lax.axis_index("s")

        zvec = jnp.zeros((16,), jnp.float32)

        def zrow(i, carry):
            for k in range(W // 16):
                bufs[0, i, pl.ds(k * 16, 16)] = zvec
            return carry

        lax.fori_loop(0, UR, zrow, 0)
        r0 = s * zr
        off = 0
        while off < zr:
            step = min(UR, zr - off)
            pltpu.sync_copy(bufs.at[0, pl.ds(0, step)],
                            acc.at[pl.ds(r0 + off, step)])
            off += step
        plsc.subcore_barrier()

        row_base = (c * NS + s) * cpt * idx_rows

        def chunk(i, carry):
            rb = row_base + i * idx_rows
            pltpu.sync_copy(src_hbm.at[pl.ds(rb, idx_rows)], srcv)
            pltpu.sync_copy(dst_hbm.at[pl.ds(rb, idx_rows)], dstv)
            gcp = {}
            scp = {}
            gcp[0] = pltpu.async_copy(y_hbm.at[srcv.at[0]], bufs.at[0],
                                      gsems[0])
            gcp[1] = pltpu.async_copy(y_hbm.at[srcv.at[1]], bufs.at[1],
                                      gsems[1])
            for u in range(idx_rows):
                b = u % 2
                gcp[u].wait()
                scp[u] = pltpu.async_copy(bufs.at[b], acc.at[dstv.at[u]],
                                          ssems[b], add=True)
                if u + 2 < idx_rows:
                    scp[u].wait()
                    gcp[u + 2] = pltpu.async_copy(
                        y_hbm.at[srcv.at[u + 2]], bufs.at[b], gsems[b])
            scp[idx_rows - 2].wait()
            scp[idx_rows - 1].wait()
            return carry

        lax.fori_loop(0, cpt, chunk, 0)
        plsc.subcore_barrier()

        pltpu.sync_copy(acc.at[pl.ds(r0, zr)],
                        out_hbm.at[pl.ds(c * np_rows + r0, zr)])

    return agg, np_rows


def _prep_body(x_ref, wcat_ref, y_ref, r_ref):
    n = y_ref.shape[0]
    hid = r_ref.shape[1]
    out = lax.dot_general(x_ref[...], wcat_ref[...],
                          (((1,), (0,)), ((), ())),
                          preferred_element_type=jnp.float32)
    col = lax.broadcasted_iota(jnp.int32, (n, W), 1)
    y_ref[...] = out[:, :W] + jnp.where(col == hid, 1.0, 0.0)
    r_ref[...] = out[:, W:]


def _mid_body(pa_ref, pb_ref, r_ref, b1_ref, g1_ref, be1_ref, w2_ref,
              yr2_ref, ci_ref):
    hid = r_ref.shape[1]
    p = pa_ref[...] + pb_ref[...]
    ssum = p[:, :hid]
    cnt = p[:, hid:hid + 1]
    cclip = jnp.maximum(cnt, 1.0)
    h = ssum / cclip + b1_ref[...][None, :] + r_ref[...]
    mu = jnp.mean(h, axis=0, keepdims=True)
    var = jnp.mean((h - mu) ** 2, axis=0, keepdims=True)
    hn = jnp.maximum(
        g1_ref[...][None, :] * (h - mu) / jnp.sqrt(var + 1e-5)
        + be1_ref[...][None, :], 0.0)
    yr2_ref[...] = lax.dot_general(hn, w2_ref[...], (((1,), (0,)), ((), ())),
                                   preferred_element_type=jnp.float32)
    ci_ref[...] = cclip


def _fin_body(pa_ref, pb_ref, yr2_ref, ci_ref, b2_ref, g2_ref, be2_ref,
              o_ref):
    hid = o_ref.shape[1]
    ssum = pa_ref[:, :hid] + pb_ref[:, :hid]
    r2 = yr2_ref[...][:, hid:]
    h = ssum / ci_ref[...] + b2_ref[...][None, :] + r2
    mu = jnp.mean(h, axis=0, keepdims=True)
    var = jnp.mean((h - mu) ** 2, axis=0, keepdims=True)
    hn = jnp.maximum(
        g2_ref[...][None, :] * (h - mu) / jnp.sqrt(var + 1e-5)
        + be2_ref[...][None, :], 0.0)
    o_ref[...] = jnp.max(hn, axis=0, keepdims=True)


def kernel(x, edge_index, Wl1, Wr1, b1, Wl2, Wr2, b2, g1, beta1, g2, beta2):
    n, in_dim = x.shape
    hid = Wl1.shape[0]
    e = edge_index.shape[1]

    e_pad = -(-e // (NW * IDXCH)) * (NW * IDXCH)
    pad = e_pad - e
    src = jnp.concatenate(
        [edge_index[0], jnp.zeros((pad,), jnp.int32)]).reshape(-1, IPR)
    dst = jnp.concatenate(
        [edge_index[1], jnp.full((pad,), n, jnp.int32)]).reshape(-1, IPR)

    w1cat = jnp.concatenate(
        [Wl1.T, jnp.zeros((in_dim, W - hid), jnp.float32), Wr1.T], axis=1)
    yaug, r1 = pl.pallas_call(
        _prep_body,
        out_shape=[jax.ShapeDtypeStruct((n, W), jnp.float32),
                   jax.ShapeDtypeStruct((n, hid), jnp.float32)],
    )(x, w1cat)

    agg, np_rows = _make_agg(n, e_pad)
    part1 = agg(src, dst, yaug)
    p1a = lax.slice(part1, (0, 0), (n, W))
    p1b = lax.slice(part1, (np_rows, 0), (np_rows + n, W))

    w2cat = jnp.concatenate([Wl2.T, Wr2.T], axis=1)
    yr2, ci = pl.pallas_call(
        _mid_body,
        out_shape=[jax.ShapeDtypeStruct((n, W), jnp.float32),
                   jax.ShapeDtypeStruct((n, 1), jnp.float32)],
    )(p1a, p1b, r1, b1, g1, beta1, w2cat)

    part2 = agg(src, dst, yr2)
    p2a = lax.slice(part2, (0, 0), (n, W))
    p2b = lax.slice(part2, (np_rows, 0), (np_rows + n, W))

    o = pl.pallas_call(
        _fin_body,
        out_shape=jax.ShapeDtypeStruct((1, hid), jnp.float32),
    )(p2a, p2b, yr2, ci, b2, g2, beta2)
    return o.reshape((hid,))


# speedup vs baseline: 1.0801x; 1.0025x over previous
import functools

import jax
import jax.numpy as jnp
from jax import lax
from jax.experimental import pallas as pl
from jax.experimental.pallas import tpu as pltpu
from jax.experimental.pallas import tpu_sc as plsc

NC = 2
NS = 16
NW = NC * NS
IPR = 128
IDXCH = 2048
W = 128


@functools.lru_cache(maxsize=None)
def _make_agg(n_nodes, e_pad):
    total_chunks = e_pad // IDXCH
    cpt = total_chunks // NW
    idx_rows = IDXCH // IPR
    NB = 2
    UR = 128
    upc = IDXCH // UR
    slab = (-(-(n_nodes + 1) // NS) + 7) // 8 * 8
    np_rows = NS * slab
    zr = slab
    mesh = plsc.VectorSubcoreMesh(core_axis_name="c", subcore_axis_name="s")

    @functools.partial(
        pl.kernel,
        out_type=jax.ShapeDtypeStruct((NC * np_rows, W), jnp.float32),
        mesh=mesh,
        scratch_types=[
            pltpu.VMEM((idx_rows, IPR), jnp.int32),
            pltpu.VMEM((idx_rows, IPR), jnp.int32),
            pltpu.VMEM((NB, UR, W), jnp.float32),
            pltpu.VMEM_SHARED((np_rows, W), jnp.float32),
            pltpu.SemaphoreType.DMA,
            pltpu.SemaphoreType.DMA,
            pltpu.SemaphoreType.DMA,
            pltpu.SemaphoreType.DMA,
        ],
    )
    def agg(src_hbm, dst_hbm, y_hbm, out_hbm, srcv, dstv, bufs, acc,
            gs0, gs1, ss0, ss1):
        gsems = (gs0, gs1)
        ssems = (ss0, ss1)
        c = lax.axis_index("c")
        s = lax.axis_index("s")
        row_base = (c * NS + s) * cpt * idx_rows

        ip0 = pltpu.async_copy(src_hbm.at[pl.ds(row_base, idx_rows)], srcv,
                               ssems[0])
        ip1 = pltpu.async_copy(dst_hbm.at[pl.ds(row_base, idx_rows)], dstv,
                               ssems[1])

        zvec = jnp.zeros((16,), jnp.float32)

        def zrow(i, carry):
            for k in range(W // 16):
                bufs[1, i, pl.ds(k * 16, 16)] = zvec
            return carry

        lax.fori_loop(0, UR, zrow, 0)
        r0 = s * zr
        off = 0
        while off < zr:
            step = min(UR, zr - off)
            pltpu.sync_copy(bufs.at[1, pl.ds(0, step)],
                            acc.at[pl.ds(r0 + off, step)])
            off += step

        ip0.wait()
        ip1.wait()
        pre0 = pltpu.async_copy(y_hbm.at[srcv.at[0]], bufs.at[0], gsems[0])
        pre1 = pltpu.async_copy(y_hbm.at[srcv.at[1]], bufs.at[1], gsems[1])
        plsc.subcore_barrier()

        def chunk_body(i, primed):
            if not primed:
                rb = row_base + i * idx_rows
                pltpu.sync_copy(src_hbm.at[pl.ds(rb, idx_rows)], srcv)
                pltpu.sync_copy(dst_hbm.at[pl.ds(rb, idx_rows)], dstv)
            gcp = {}
            scp = {}
            if primed:
                gcp[0], gcp[1] = pre0, pre1
            else:
                gcp[0] = pltpu.async_copy(y_hbm.at[srcv.at[0]], bufs.at[0],
                                          gsems[0])
                gcp[1] = pltpu.async_copy(y_hbm.at[srcv.at[1]], bufs.at[1],
                                          gsems[1])
            for u in range(idx_rows):
                b = u % 2
                gcp[u].wait()
                scp[u] = pltpu.async_copy(bufs.at[b], acc.at[dstv.at[u]],
                                          ssems[b], add=True)
                if u + 2 < idx_rows:
                    scp[u].wait()
                    gcp[u + 2] = pltpu.async_copy(
                        y_hbm.at[srcv.at[u + 2]], bufs.at[b], gsems[b])
            scp[idx_rows - 2].wait()
            scp[idx_rows - 1].wait()

        chunk_body(0, True)
        lax.fori_loop(1, cpt, lambda i, cr: (chunk_body(i, False), cr)[1], 0)
        plsc.subcore_barrier()

        pltpu.sync_copy(acc.at[pl.ds(r0, zr)],
                        out_hbm.at[pl.ds(c * np_rows + r0, zr)])

    return agg, np_rows


def _prep_body(x_ref, wcat_ref, y_ref, r_ref):
    n = y_ref.shape[0]
    hid = r_ref.shape[1]
    out = lax.dot_general(x_ref[...], wcat_ref[...],
                          (((1,), (0,)), ((), ())),
                          preferred_element_type=jnp.float32)
    col = lax.broadcasted_iota(jnp.int32, (n, W), 1)
    y_ref[...] = out[:, :W] + jnp.where(col == hid, 1.0, 0.0)
    r_ref[...] = out[:, W:]


def _mid_body(pa_ref, pb_ref, r_ref, b1_ref, g1_ref, be1_ref, w2_ref,
              yr2_ref, ci_ref):
    hid = r_ref.shape[1]
    p = pa_ref[...] + pb_ref[...]
    ssum = p[:, :hid]
    cnt = p[:, hid:hid + 1]
    cclip = jnp.maximum(cnt, 1.0)
    h = ssum / cclip + b1_ref[...][None, :] + r_ref[...]
    mu = jnp.mean(h, axis=0, keepdims=True)
    var = jnp.mean((h - mu) ** 2, axis=0, keepdims=True)
    hn = jnp.maximum(
        g1_ref[...][None, :] * (h - mu) / jnp.sqrt(var + 1e-5)
        + be1_ref[...][None, :], 0.0)
    yr2_ref[...] = lax.dot_general(hn, w2_ref[...], (((1,), (0,)), ((), ())),
                                   preferred_element_type=jnp.float32)
    ci_ref[...] = cclip


def _fin_body(pa_ref, pb_ref, yr2_ref, ci_ref, b2_ref, g2_ref, be2_ref,
              o_ref):
    hid = o_ref.shape[1]
    ssum = pa_ref[:, :hid] + pb_ref[:, :hid]
    r2 = yr2_ref[...][:, hid:]
    h = ssum / ci_ref[...] + b2_ref[...][None, :] + r2
    mu = jnp.mean(h, axis=0, keepdims=True)
    var = jnp.mean((h - mu) ** 2, axis=0, keepdims=True)
    hn = jnp.maximum(
        g2_ref[...][None, :] * (h - mu) / jnp.sqrt(var + 1e-5)
        + be2_ref[...][None, :], 0.0)
    o_ref[...] = jnp.max(hn, axis=0, keepdims=True)


def kernel(x, edge_index, Wl1, Wr1, b1, Wl2, Wr2, b2, g1, beta1, g2, beta2):
    n, in_dim = x.shape
    hid = Wl1.shape[0]
    e = edge_index.shape[1]

    e_pad = -(-e // (NW * IDXCH)) * (NW * IDXCH)
    pad = e_pad - e
    src = jnp.concatenate(
        [edge_index[0], jnp.zeros((pad,), jnp.int32)]).reshape(-1, IPR)
    dst = jnp.concatenate(
        [edge_index[1], jnp.full((pad,), n, jnp.int32)]).reshape(-1, IPR)

    w1cat = jnp.concatenate(
        [Wl1.T, jnp.zeros((in_dim, W - hid), jnp.float32), Wr1.T], axis=1)
    yaug, r1 = pl.pallas_call(
        _prep_body,
        out_shape=[jax.ShapeDtypeStruct((n, W), jnp.float32),
                   jax.ShapeDtypeStruct((n, hid), jnp.float32)],
    )(x, w1cat)

    agg, np_rows = _make_agg(n, e_pad)
    part1 = agg(src, dst, yaug)
    p1a = lax.slice(part1, (0, 0), (n, W))
    p1b = lax.slice(part1, (np_rows, 0), (np_rows + n, W))

    w2cat = jnp.concatenate([Wl2.T, Wr2.T], axis=1)
    yr2, ci = pl.pallas_call(
        _mid_body,
        out_shape=[jax.ShapeDtypeStruct((n, W), jnp.float32),
                   jax.ShapeDtypeStruct((n, 1), jnp.float32)],
    )(p1a, p1b, r1, b1, g1, beta1, w2cat)

    part2 = agg(src, dst, yr2)
    p2a = lax.slice(part2, (0, 0), (n, W))
    p2b = lax.slice(part2, (np_rows, 0), (np_rows + n, W))

    o = pl.pallas_call(
        _fin_body,
        out_shape=jax.ShapeDtypeStruct((1, hid), jnp.float32),
    )(p2a, p2b, yr2, ci, b2, g2, beta2)
    return o.reshape((hid,))


# speedup vs baseline: 1.1094x; 1.0271x over previous
import functools

import jax
import jax.numpy as jnp
from jax import lax
from jax.experimental import pallas as pl
from jax.experimental.pallas import tpu as pltpu
from jax.experimental.pallas import tpu_sc as plsc

NC = 2
NS = 16
NW = NC * NS
IPR = 128
IDXCH = 2048
W = 128


@functools.lru_cache(maxsize=None)
def _make_agg(n_nodes, e_pad):
    total_chunks = e_pad // IDXCH
    cpt = total_chunks // NW
    idx_rows = IDXCH // IPR
    NB = 2
    UR = 128
    upc = IDXCH // UR
    slab = (-(-(n_nodes + 1) // NS) + 7) // 8 * 8
    np_rows = NS * slab
    zr = slab
    mesh = plsc.VectorSubcoreMesh(core_axis_name="c", subcore_axis_name="s")

    @functools.partial(
        pl.kernel,
        out_type=jax.ShapeDtypeStruct((NC * np_rows, W), jnp.float32),
        mesh=mesh,
        scratch_types=[
            pltpu.VMEM((2, idx_rows, IPR), jnp.int32),
            pltpu.VMEM((2, idx_rows, IPR), jnp.int32),
            pltpu.VMEM((NB, UR, W), jnp.float32),
            pltpu.VMEM_SHARED((np_rows, W), jnp.float32),
            pltpu.SemaphoreType.DMA,
            pltpu.SemaphoreType.DMA,
            pltpu.SemaphoreType.DMA,
            pltpu.SemaphoreType.DMA,
            pltpu.SemaphoreType.DMA,
            pltpu.SemaphoreType.DMA,
        ],
    )
    def agg(src_hbm, dst_hbm, y_hbm, out_hbm, srcv, dstv, bufs, acc,
            gs0, gs1, ss0, ss1, is0, is1):
        gsems = (gs0, gs1)
        ssems = (ss0, ss1)
        isems = (is0, is1)
        c = lax.axis_index("c")
        s = lax.axis_index("s")
        row_base = (c * NS + s) * cpt * idx_rows

        def idx_fetch(i):
            k = i % 2
            rb = row_base + i * idx_rows
            return (pltpu.async_copy(src_hbm.at[pl.ds(rb, idx_rows)],
                                     srcv.at[k], isems[k]),
                    pltpu.async_copy(dst_hbm.at[pl.ds(rb, idx_rows)],
                                     dstv.at[k], isems[k]))

        ipend = {0: idx_fetch(0)}

        zvec = jnp.zeros((16,), jnp.float32)

        def zrow(i, carry):
            for k in range(W // 16):
                bufs[1, i, pl.ds(k * 16, 16)] = zvec
            return carry

        lax.fori_loop(0, UR, zrow, 0)
        r0 = s * zr
        off = 0
        while off < zr:
            step = min(UR, zr - off)
            pltpu.sync_copy(bufs.at[1, pl.ds(0, step)],
                            acc.at[pl.ds(r0 + off, step)])
            off += step

        ipend[0][0].wait()
        ipend[0][1].wait()
        if cpt > 1:
            ipend[1] = idx_fetch(1)
        gcp = {}
        scp = {}
        gcp[0] = pltpu.async_copy(y_hbm.at[srcv.at[0, 0]], bufs.at[0],
                                  gsems[0])
        gcp[1] = pltpu.async_copy(y_hbm.at[srcv.at[0, 1]], bufs.at[1],
                                  gsems[1])
        plsc.subcore_barrier()

        U = cpt * idx_rows
        for g in range(U):
            j, u = divmod(g, idx_rows)
            b = g % 2
            gcp[g].wait()
            scp[g] = pltpu.async_copy(bufs.at[b], acc.at[dstv.at[j % 2, u]],
                                      ssems[b], add=True)
            if u == 0 and 1 <= j and j + 1 < cpt:
                ipend[j + 1] = idx_fetch(j + 1)
            nxt = g + 2
            if nxt < U:
                scp[g].wait()
                nj, nu = divmod(nxt, idx_rows)
                if nu == 0:
                    ipend[nj][0].wait()
                    ipend[nj][1].wait()
                gcp[nxt] = pltpu.async_copy(y_hbm.at[srcv.at[nj % 2, nu]],
                                            bufs.at[b], gsems[b])
        scp[U - 2].wait()
        scp[U - 1].wait()
        plsc.subcore_barrier()

        pltpu.sync_copy(acc.at[pl.ds(r0, zr)],
                        out_hbm.at[pl.ds(c * np_rows + r0, zr)])

    return agg, np_rows


def _prep_body(x_ref, wcat_ref, y_ref, r_ref):
    n = y_ref.shape[0]
    hid = r_ref.shape[1]
    out = lax.dot_general(x_ref[...], wcat_ref[...],
                          (((1,), (0,)), ((), ())),
                          preferred_element_type=jnp.float32)
    col = lax.broadcasted_iota(jnp.int32, (n, W), 1)
    y_ref[...] = out[:, :W] + jnp.where(col == hid, 1.0, 0.0)
    r_ref[...] = out[:, W:]


def _mid_body(pa_ref, pb_ref, r_ref, b1_ref, g1_ref, be1_ref, w2_ref,
              yr2_ref, ci_ref):
    hid = r_ref.shape[1]
    p = pa_ref[...] + pb_ref[...]
    ssum = p[:, :hid]
    cnt = p[:, hid:hid + 1]
    cclip = jnp.maximum(cnt, 1.0)
    h = ssum / cclip + b1_ref[...][None, :] + r_ref[...]
    mu = jnp.mean(h, axis=0, keepdims=True)
    var = jnp.mean((h - mu) ** 2, axis=0, keepdims=True)
    hn = jnp.maximum(
        g1_ref[...][None, :] * (h - mu) / jnp.sqrt(var + 1e-5)
        + be1_ref[...][None, :], 0.0)
    yr2_ref[...] = lax.dot_general(hn, w2_ref[...], (((1,), (0,)), ((), ())),
                                   preferred_element_type=jnp.float32)
    ci_ref[...] = cclip


def _fin_body(pa_ref, pb_ref, yr2_ref, ci_ref, b2_ref, g2_ref, be2_ref,
              o_ref):
    hid = o_ref.shape[1]
    ssum = pa_ref[:, :hid] + pb_ref[:, :hid]
    r2 = yr2_ref[...][:, hid:]
    h = ssum / ci_ref[...] + b2_ref[...][None, :] + r2
    mu = jnp.mean(h, axis=0, keepdims=True)
    var = jnp.mean((h - mu) ** 2, axis=0, keepdims=True)
    hn = jnp.maximum(
        g2_ref[...][None, :] * (h - mu) / jnp.sqrt(var + 1e-5)
        + be2_ref[...][None, :], 0.0)
    o_ref[...] = jnp.max(hn, axis=0, keepdims=True)


def kernel(x, edge_index, Wl1, Wr1, b1, Wl2, Wr2, b2, g1, beta1, g2, beta2):
    n, in_dim = x.shape
    hid = Wl1.shape[0]
    e = edge_index.shape[1]

    e_pad = -(-e // (NW * IDXCH)) * (NW * IDXCH)
    pad = e_pad - e
    src = jnp.concatenate(
        [edge_index[0], jnp.zeros((pad,), jnp.int32)]).reshape(-1, IPR)
    dst = jnp.concatenate(
        [edge_index[1], jnp.full((pad,), n, jnp.int32)]).reshape(-1, IPR)

    w1cat = jnp.concatenate(
        [Wl1.T, jnp.zeros((in_dim, W - hid), jnp.float32), Wr1.T], axis=1)
    yaug, r1 = pl.pallas_call(
        _prep_body,
        out_shape=[jax.ShapeDtypeStruct((n, W), jnp.float32),
                   jax.ShapeDtypeStruct((n, hid), jnp.float32)],
    )(x, w1cat)

    agg, np_rows = _make_agg(n, e_pad)
    part1 = agg(src, dst, yaug)
    p1a = lax.slice(part1, (0, 0), (n, W))
    p1b = lax.slice(part1, (np_rows, 0), (np_rows + n, W))

    w2cat = jnp.concatenate([Wl2.T, Wr2.T], axis=1)
    yr2, ci = pl.pallas_call(
        _mid_body,
        out_shape=[jax.ShapeDtypeStruct((n, W), jnp.float32),
                   jax.ShapeDtypeStruct((n, 1), jnp.float32)],
    )(p1a, p1b, r1, b1, g1, beta1, w2cat)

    part2 = agg(src, dst, yr2)
    p2a = lax.slice(part2, (0, 0), (n, W))
    p2b = lax.slice(part2, (np_rows, 0), (np_rows + n, W))

    o = pl.pallas_call(
        _fin_body,
        out_shape=jax.ShapeDtypeStruct((1, hid), jnp.float32),
    )(p2a, p2b, yr2, ci, b2, g2, beta2)
    return o.reshape((hid,))
